# baseline (speedup 1.0000x reference)
import functools

import jax
import jax.numpy as jnp
from jax import lax
from jax.experimental import pallas as pl
from jax.experimental.pallas import tpu as pltpu
from jax.experimental.pallas import tpu_sc as plsc

B = 16384
F = 512
D = 32
NC = 2
NS = 16
NW = NC * NS
BPW = B // NW

BLK = 1024
BLK2 = 4096


def _sc_gather_body(uidx, pidx, nidx, uemb, iemb, ue_out, pe_out, ne_out,
                    idx_v, rows_v, sem):
    wid = lax.axis_index("s") * NC + lax.axis_index("c")
    base = wid * BPW

    def do(idx_hbm, table, out_hbm):
        pltpu.sync_copy(idx_hbm.at[pl.ds(base, BPW)], idx_v)
        pltpu.async_copy(table.at[idx_v], rows_v, sem).wait()
        pltpu.sync_copy(rows_v, out_hbm.at[pl.ds(base, BPW)])

    do(uidx, uemb, ue_out)
    do(pidx, iemb, pe_out)
    do(nidx, iemb, ne_out)


def _sc_gather(uidx, pidx, nidx, uemb, iemb):
    mesh = plsc.VectorSubcoreMesh(core_axis_name="c", subcore_axis_name="s")
    out = jax.ShapeDtypeStruct((B, D), jnp.bfloat16)
    fn = functools.partial(
        pl.kernel,
        mesh=mesh,
        out_type=(out, out, out),
        scratch_types=[
            pltpu.VMEM((BPW,), jnp.int32),
            pltpu.VMEM((BPW, D), jnp.bfloat16),
            pltpu.SemaphoreType.DMA,
        ],
        compiler_params=pltpu.CompilerParams(use_tc_tiling_on_sc=False),
    )(_sc_gather_body)
    return fn(uidx, pidx, nidx, uemb, iemb)


def _tca_body(uf, pf, nf, umap, imap, w1b, hp_out, hn_out):
    un = (uf[...] - 2.5) * 0.4
    pn = (pf[...] - 2.5) * 0.4
    nn = (nf[...] - 2.5) * 0.4
    um = jnp.dot(un, umap[...], preferred_element_type=jnp.float32)
    pm = jnp.dot(pn, imap[...], preferred_element_type=jnp.float32)
    nm = jnp.dot(nn, imap[...], preferred_element_type=jnp.float32)

    c_last = (((1,), (1,)), ((), ()))
    hp_out[...] = lax.dot_general(um * pm, w1b[...], c_last,
                                  preferred_element_type=jnp.float32)
    hn_out[...] = lax.dot_general(um * nm, w1b[...], c_last,
                                  preferred_element_type=jnp.float32)


def _tca(uf, pf, nf, umap, imap, w1b):
    grid = B // BLK
    feat_spec = pl.BlockSpec((BLK, F), lambda i: (i, 0))

    def full(shape):
        return pl.BlockSpec(shape, lambda i: tuple(0 for _ in shape))

    out = jax.ShapeDtypeStruct((B, 10), jnp.float32)
    return pl.pallas_call(
        _tca_body,
        grid=(grid,),
        in_specs=[feat_spec, feat_spec, feat_spec,
                  full((F, D)), full((F, D)), full((10, D))],
        out_specs=(pl.BlockSpec((BLK, 10), lambda i: (i, 0)),
                   pl.BlockSpec((BLK, 10), lambda i: (i, 0))),
        out_shape=(out, out),
    )(uf, pf, nf, umap, imap, w1b)


def _tcb_body(ue, pe, ne, hfp, hfn, w1a, b1, w2, out):
    i = pl.program_id(0)
    uev = ue[...].astype(jnp.float32)
    uip = uev * pe[...].astype(jnp.float32)
    uin = uev * ne[...].astype(jnp.float32)

    c_last = (((1,), (1,)), ((), ()))
    hp = lax.dot_general(uip, w1a[...], c_last,
                         preferred_element_type=jnp.float32)
    hp = jnp.maximum(hp + hfp[...] + b1[...], 0.0)
    hn = lax.dot_general(uin, w1a[...], c_last,
                         preferred_element_type=jnp.float32)
    hn = jnp.maximum(hn + hfn[...] + b1[...], 0.0)

    d = lax.dot_general(hp - hn, w2[...], c_last,
                        preferred_element_type=jnp.float32)
    part = jnp.sum(jnp.minimum(d, 0.0) - jnp.log1p(jnp.exp(-jnp.abs(d))))

    @pl.when(i == 0)
    def _():
        out[0, 0] = 0.0

    out[0, 0] += part


def _tcb(ue, pe, ne, hfp, hfn, w1a, b1, w2):
    grid = B // BLK2
    emb_spec = pl.BlockSpec((BLK2, D), lambda i: (i, 0))
    h_spec = pl.BlockSpec((BLK2, 10), lambda i: (i, 0))

    def full(shape):
        return pl.BlockSpec(shape, lambda i: tuple(0 for _ in shape))

    return pl.pallas_call(
        _tcb_body,
        grid=(grid,),
        in_specs=[emb_spec, emb_spec, emb_spec, h_spec, h_spec,
                  full((10, D)), full((1, 10)), full((1, 10))],
        out_specs=pl.BlockSpec((1, 1), lambda i: (0, 0),
                               memory_space=pltpu.SMEM),
        out_shape=jax.ShapeDtypeStruct((1, 1), jnp.float32),
    )(ue, pe, ne, hfp, hfn, w1a, b1, w2)


def kernel(user_batch, user_feature_batch, pos_item_batch,
           pos_item_feature_batch, neg_item_batch, neg_item_feature_batch,
           user_emb, item_emb, user_map, item_map,
           fc1_w, fc1_b, fc2_w, fc2_b):
    uidx = user_batch.astype(jnp.int32)
    pidx = pos_item_batch.astype(jnp.int32)
    nidx = neg_item_batch.astype(jnp.int32)

    ue, pe, ne = _sc_gather(uidx, pidx, nidx,
                            user_emb.astype(jnp.bfloat16),
                            item_emb.astype(jnp.bfloat16))
    hfp, hfn = _tca(user_feature_batch, pos_item_feature_batch,
                    neg_item_feature_batch, user_map, item_map,
                    fc1_w[:, D:])
    total = _tcb(ue, pe, ne, hfp, hfn, fc1_w[:, :D],
                 fc1_b.reshape(1, 10), fc2_w)
    return -total[0, 0] / B


# speedup vs baseline: 1.1563x; 1.1563x over previous
import functools

import jax
import jax.numpy as jnp
from jax import lax
from jax.experimental import pallas as pl
from jax.experimental.pallas import tpu as pltpu
from jax.experimental.pallas import tpu_sc as plsc

B = 16384
F = 512
D = 32
NC = 2
NS = 16
NW = NC * NS
BPW = B // NW

BLK = 1024
BLK2 = 4096


def _sc_gather_body(uidx, pidx, nidx, uemb, iemb, ue_out, pe_out, ne_out,
                    idx_v, rows_v, sem):
    wid = lax.axis_index("s") * NC + lax.axis_index("c")
    base = wid * BPW

    def do(idx_hbm, table, out_hbm):
        pltpu.sync_copy(idx_hbm.at[pl.ds(base, BPW)], idx_v)
        pltpu.async_copy(table.at[idx_v], rows_v, sem).wait()
        pltpu.sync_copy(rows_v, out_hbm.at[pl.ds(base, BPW)])

    do(uidx, uemb, ue_out)
    do(pidx, iemb, pe_out)
    do(nidx, iemb, ne_out)


def _sc_gather(uidx, pidx, nidx, uemb, iemb):
    mesh = plsc.VectorSubcoreMesh(core_axis_name="c", subcore_axis_name="s")
    out = jax.ShapeDtypeStruct((B, D), jnp.float32)
    fn = functools.partial(
        pl.kernel,
        mesh=mesh,
        out_type=(out, out, out),
        scratch_types=[
            pltpu.VMEM((BPW,), jnp.int32),
            pltpu.VMEM((BPW, D), jnp.float32),
            pltpu.SemaphoreType.DMA,
        ],
        compiler_params=pltpu.CompilerParams(use_tc_tiling_on_sc=False),
    )(_sc_gather_body)
    return fn(uidx, pidx, nidx, uemb, iemb)


def _tca_body(uf, pf, nf, umap, imap, w1b, hp_out, hn_out):
    un = (uf[...] - 2.5) * 0.4
    pn = (pf[...] - 2.5) * 0.4
    nn = (nf[...] - 2.5) * 0.4
    um = jnp.dot(un, umap[...], preferred_element_type=jnp.float32)
    pm = jnp.dot(pn, imap[...], preferred_element_type=jnp.float32)
    nm = jnp.dot(nn, imap[...], preferred_element_type=jnp.float32)

    c_last = (((1,), (1,)), ((), ()))
    hp_out[...] = lax.dot_general(um * pm, w1b[...], c_last,
                                  preferred_element_type=jnp.float32)
    hn_out[...] = lax.dot_general(um * nm, w1b[...], c_last,
                                  preferred_element_type=jnp.float32)


def _tca(uf, pf, nf, umap, imap, w1b):
    grid = B // BLK
    feat_spec = pl.BlockSpec((BLK, F), lambda i: (i, 0))

    def full(shape):
        return pl.BlockSpec(shape, lambda i: tuple(0 for _ in shape))

    out = jax.ShapeDtypeStruct((B, 10), jnp.float32)
    return pl.pallas_call(
        _tca_body,
        grid=(grid,),
        in_specs=[feat_spec, feat_spec, feat_spec,
                  full((F, D)), full((F, D)), full((10, D))],
        out_specs=(pl.BlockSpec((BLK, 10), lambda i: (i, 0)),
                   pl.BlockSpec((BLK, 10), lambda i: (i, 0))),
        out_shape=(out, out),
    )(uf, pf, nf, umap, imap, w1b)


def _tcb_body(ue, pe, ne, hfp, hfn, w1a, b1, w2, out):
    i = pl.program_id(0)
    uip = ue[...] * pe[...]
    uin = ue[...] * ne[...]

    c_last = (((1,), (1,)), ((), ()))
    hp = lax.dot_general(uip, w1a[...], c_last,
                         preferred_element_type=jnp.float32)
    hp = jnp.maximum(hp + hfp[...] + b1[...], 0.0)
    hn = lax.dot_general(uin, w1a[...], c_last,
                         preferred_element_type=jnp.float32)
    hn = jnp.maximum(hn + hfn[...] + b1[...], 0.0)

    d = lax.dot_general(hp - hn, w2[...], c_last,
                        preferred_element_type=jnp.float32)
    part = jnp.sum(jnp.minimum(d, 0.0) - jnp.log1p(jnp.exp(-jnp.abs(d))))

    @pl.when(i == 0)
    def _():
        out[0, 0] = 0.0

    out[0, 0] += part


def _tcb(ue, pe, ne, hfp, hfn, w1a, b1, w2):
    grid = B // BLK2
    emb_spec = pl.BlockSpec((BLK2, D), lambda i: (i, 0))
    h_spec = pl.BlockSpec((BLK2, 10), lambda i: (i, 0))

    def full(shape):
        return pl.BlockSpec(shape, lambda i: tuple(0 for _ in shape))

    return pl.pallas_call(
        _tcb_body,
        grid=(grid,),
        in_specs=[emb_spec, emb_spec, emb_spec, h_spec, h_spec,
                  full((10, D)), full((1, 10)), full((1, 10))],
        out_specs=pl.BlockSpec((1, 1), lambda i: (0, 0),
                               memory_space=pltpu.SMEM),
        out_shape=jax.ShapeDtypeStruct((1, 1), jnp.float32),
    )(ue, pe, ne, hfp, hfn, w1a, b1, w2)


def kernel(user_batch, user_feature_batch, pos_item_batch,
           pos_item_feature_batch, neg_item_batch, neg_item_feature_batch,
           user_emb, item_emb, user_map, item_map,
           fc1_w, fc1_b, fc2_w, fc2_b):
    uidx = user_batch.astype(jnp.int32)
    pidx = pos_item_batch.astype(jnp.int32)
    nidx = neg_item_batch.astype(jnp.int32)

    ue, pe, ne = _sc_gather(uidx, pidx, nidx, user_emb, item_emb)
    hfp, hfn = _tca(user_feature_batch, pos_item_feature_batch,
                    neg_item_feature_batch, user_map, item_map,
                    fc1_w[:, D:])
    total = _tcb(ue, pe, ne, hfp, hfn, fc1_w[:, :D],
                 fc1_b.reshape(1, 10), fc2_w)
    return -total[0, 0] / B


# speedup vs baseline: 1.2433x; 1.0753x over previous
import functools

import jax
import jax.numpy as jnp
from jax import lax
from jax.experimental import pallas as pl
from jax.experimental.pallas import tpu as pltpu
from jax.experimental.pallas import tpu_sc as plsc

B = 16384
F = 512
D = 32
U = 1000000
NC = 2
NS = 16
NW = NC * NS

LPW = 31360
LANE_END = 999936
CH = 1536
NCHUNK = 21
RU_CAP = 16896
RI_CAP = 33280
OUTN = 3 * B + 16
TRASH = 3 * B

BLK = 1024
BLK2 = 4096


def _sc_body(uidx, pidx, nidx, ut, it, tail_u, tail_i,
             out, hlu, hbu, hli, hbi,
             idxbuf, stg_lu, stg_bu, stg_li, stg_bi, chunk_v,
             hl_v, hb_v, pend_l, pend_b, dst_v, packed, sem):
    wid = lax.axis_index("s") * NC + lax.axis_index("c")
    start_w = wid * LPW
    end_w = jnp.minimum(start_w + LPW, LANE_END)
    ru = wid * RU_CAP
    ri = wid * RI_CAP
    iota16 = lax.iota(jnp.int32, 16)
    is_last = wid == NW - 1

    def scan_stream(stream, idx_hbm, stg_l, stg_b, hl, hb, reg, carry):
        off, flushed = carry
        for cstart in range(0, B, 2048):
            pltpu.sync_copy(idx_hbm.at[pl.ds(cstart, 2048)], idxbuf)

            def body(g, c):
                off, flushed = c
                v = idxbuf[pl.ds(pl.multiple_of(g * 16, 16), 16)]
                m = jnp.logical_and(v >= start_w, v < end_w)
                m = jnp.logical_or(
                    m, jnp.logical_and(v >= LANE_END,
                                       jnp.full((16,), is_last)))
                b = stream * B + cstart + g * 16 + iota16
                plsc.store_compressed(stg_l.at[pl.ds(off, 16)], v, mask=m)
                plsc.store_compressed(stg_b.at[pl.ds(off, 16)], b, mask=m)
                off = off + jnp.sum(m.astype(jnp.int32))
                fl = off >= 256

                @pl.when(fl)
                def _():
                    pltpu.sync_copy(stg_l.at[pl.ds(0, 256)],
                                    hl.at[pl.ds(pl.multiple_of(reg + flushed, 256), 256)])
                    pltpu.sync_copy(stg_b.at[pl.ds(0, 256)],
                                    hb.at[pl.ds(pl.multiple_of(reg + flushed, 256), 256)])
                    t1 = stg_l[pl.ds(256, 16)]
                    t2 = stg_b[pl.ds(256, 16)]
                    stg_l[pl.ds(0, 16)] = t1
                    stg_b[pl.ds(0, 16)] = t2

                off = jnp.where(fl, off - 256, off)
                flushed = jnp.where(fl, flushed + 256, flushed)
                return (off, flushed)

            off, flushed = lax.fori_loop(0, 2048 // 16, body, (off, flushed))
        return off, flushed

    offu, flu = scan_stream(0, uidx, stg_lu, stg_bu, hlu, hbu, ru, (0, 0))
    pltpu.sync_copy(stg_lu.at[pl.ds(0, 256)], hlu.at[pl.ds(pl.multiple_of(ru + flu, 256), 256)])
    pltpu.sync_copy(stg_bu.at[pl.ds(0, 256)], hbu.at[pl.ds(pl.multiple_of(ru + flu, 256), 256)])
    cnt_u = flu + offu

    offi, fli = scan_stream(1, pidx, stg_li, stg_bi, hli, hbi, ri, (0, 0))
    offi, fli = scan_stream(2, nidx, stg_li, stg_bi, hli, hbi, ri,
                            (offi, fli))
    pltpu.sync_copy(stg_li.at[pl.ds(0, 256)], hli.at[pl.ds(pl.multiple_of(ri + fli, 256), 256)])
    pltpu.sync_copy(stg_bi.at[pl.ds(0, 256)], hbi.at[pl.ds(pl.multiple_of(ri + fli, 256), 256)])
    cnt_i = fli + offi

    def extract_group(poff_valid):
        mvalid = iota16 < poff_valid
        lanes = pend_l[pl.ds(0, 16)]
        lanes = jnp.where(mvalid, lanes, 0)
        dst = jnp.where(mvalid, pend_b[pl.ds(0, 16)], TRASH)
        dst_v[pl.ds(0, 16)] = dst
        for d in range(D):
            vals = plsc.load_gather(chunk_v,
                                    [jnp.full((16,), d, jnp.int32), lanes])
            plsc.store_scatter(packed,
                               [iota16, jnp.full((16,), d, jnp.int32)], vals)
        pltpu.async_copy(packed, out.at[dst_v], sem).wait()

    def pass_b(table, tail, hl, hb, reg, cnt):
        nblk = (cnt + 511) // 512

        def chunk_iter(j, _):
            cb = start_w + j * CH
            cb2 = jnp.minimum(cb, end_w - CH)

            @pl.when(cb < end_w)
            def _():
                pltpu.sync_copy(table.at[:, pl.ds(pl.multiple_of(cb2, 128), CH)], chunk_v)

                def blk_iter(k, poff):
                    pltpu.sync_copy(hl.at[pl.ds(pl.multiple_of(reg + k * 512, 512), 512)], hl_v)
                    pltpu.sync_copy(hb.at[pl.ds(pl.multiple_of(reg + k * 512, 512), 512)], hb_v)

                    def vreg_iter(g, poff):
                        v = hl_v[pl.ds(pl.multiple_of(g * 16, 16), 16)]
                        bv = hb_v[pl.ds(pl.multiple_of(g * 16, 16), 16)]
                        pos = k * 512 + g * 16 + iota16
                        m = jnp.logical_and(pos < cnt,
                                            jnp.logical_and(v >= cb2,
                                                            v < cb2 + CH))

                        @pl.when(jnp.sum(m.astype(jnp.int32)) > 0)
                        def _():
                            plsc.store_compressed(
                                pend_l.at[pl.ds(poff, 16)], v - cb2, mask=m)
                            plsc.store_compressed(
                                pend_b.at[pl.ds(poff, 16)], bv, mask=m)

                        poff = poff + jnp.sum(m.astype(jnp.int32))

                        @pl.when(poff >= 16)
                        def _():
                            extract_group(16)
                            t1 = pend_l[pl.ds(16, 16)]
                            t2 = pend_b[pl.ds(16, 16)]
                            pend_l[pl.ds(0, 16)] = t1
                            pend_b[pl.ds(0, 16)] = t2

                        poff = jnp.where(poff >= 16, poff - 16, poff)
                        return poff

                    return lax.fori_loop(0, 32, vreg_iter, poff)

                poff = lax.fori_loop(0, nblk, blk_iter, 0)

                @pl.when(poff > 0)
                def _():
                    extract_group(poff)

            return 0

        lax.fori_loop(0, NCHUNK, chunk_iter, 0)

        @pl.when(jnp.logical_and(jnp.full((), is_last), cnt > 0))
        def _():
            pltpu.sync_copy(tail, chunk_v.at[:, pl.ds(0, 128)])

            def blk_iter(k, poff):
                pltpu.sync_copy(hl.at[pl.ds(pl.multiple_of(reg + k * 512, 512), 512)], hl_v)
                pltpu.sync_copy(hb.at[pl.ds(pl.multiple_of(reg + k * 512, 512), 512)], hb_v)

                def vreg_iter(g, poff):
                    v = hl_v[pl.ds(pl.multiple_of(g * 16, 16), 16)]
                    bv = hb_v[pl.ds(pl.multiple_of(g * 16, 16), 16)]
                    pos = k * 512 + g * 16 + iota16
                    m = jnp.logical_and(pos < cnt, v >= LANE_END)

                    @pl.when(jnp.sum(m.astype(jnp.int32)) > 0)
                    def _():
                        plsc.store_compressed(
                            pend_l.at[pl.ds(poff, 16)], v - LANE_END, mask=m)
                        plsc.store_compressed(
                            pend_b.at[pl.ds(poff, 16)], bv, mask=m)

                    poff = poff + jnp.sum(m.astype(jnp.int32))

                    @pl.when(poff >= 16)
                    def _():
                        extract_group(16)
                        t1 = pend_l[pl.ds(16, 16)]
                        t2 = pend_b[pl.ds(16, 16)]
                        pend_l[pl.ds(0, 16)] = t1
                        pend_b[pl.ds(0, 16)] = t2

                    poff = jnp.where(poff >= 16, poff - 16, poff)
                    return poff

                return lax.fori_loop(0, 32, vreg_iter, poff)

            poff = lax.fori_loop(0, (cnt + 511) // 512, blk_iter, 0)

            @pl.when(poff > 0)
            def _():
                extract_group(poff)

    pass_b(ut, tail_u, hlu, hbu, ru, cnt_u)
    pass_b(it, tail_i, hli, hbi, ri, cnt_i)


def _sc_gather(uidx, pidx, nidx, ut, it, tail_u, tail_i):
    mesh = plsc.VectorSubcoreMesh(core_axis_name="c", subcore_axis_name="s")
    fn = functools.partial(
        pl.kernel,
        mesh=mesh,
        out_type=(
            jax.ShapeDtypeStruct((OUTN, 128), jnp.float32),
            jax.ShapeDtypeStruct((NW * RU_CAP,), jnp.int32),
            jax.ShapeDtypeStruct((NW * RU_CAP,), jnp.int32),
            jax.ShapeDtypeStruct((NW * RI_CAP,), jnp.int32),
            jax.ShapeDtypeStruct((NW * RI_CAP,), jnp.int32),
        ),
        scratch_types=[
            pltpu.VMEM((2048,), jnp.int32),
            pltpu.VMEM((288,), jnp.int32),
            pltpu.VMEM((288,), jnp.int32),
            pltpu.VMEM((288,), jnp.int32),
            pltpu.VMEM((288,), jnp.int32),
            pltpu.VMEM((D, CH), jnp.float32),
            pltpu.VMEM((512,), jnp.int32),
            pltpu.VMEM((512,), jnp.int32),
            pltpu.VMEM((48,), jnp.int32),
            pltpu.VMEM((48,), jnp.int32),
            pltpu.VMEM((16,), jnp.int32),
            pltpu.VMEM((16, 128), jnp.float32),
            pltpu.SemaphoreType.DMA,
        ],
        compiler_params=pltpu.CompilerParams(needs_layout_passes=False),
    )(_sc_body)
    return fn(uidx, pidx, nidx, ut, it, tail_u, tail_i)


def _tca_body(uf, pf, nf, umap, imap, w1b, hp_out, hn_out):
    un = (uf[...] - 2.5) * 0.4
    pn = (pf[...] - 2.5) * 0.4
    nn = (nf[...] - 2.5) * 0.4
    um = jnp.dot(un, umap[...], preferred_element_type=jnp.float32)
    pm = jnp.dot(pn, imap[...], preferred_element_type=jnp.float32)
    nm = jnp.dot(nn, imap[...], preferred_element_type=jnp.float32)

    c_last = (((1,), (1,)), ((), ()))
    hp_out[...] = lax.dot_general(um * pm, w1b[...], c_last,
                                  preferred_element_type=jnp.float32)
    hn_out[...] = lax.dot_general(um * nm, w1b[...], c_last,
                                  preferred_element_type=jnp.float32)


def _tca(uf, pf, nf, umap, imap, w1b):
    grid = B // BLK
    feat_spec = pl.BlockSpec((BLK, F), lambda i: (i, 0))

    def full(shape):
        return pl.BlockSpec(shape, lambda i: tuple(0 for _ in shape))

    out = jax.ShapeDtypeStruct((B, 10), jnp.float32)
    return pl.pallas_call(
        _tca_body,
        grid=(grid,),
        in_specs=[feat_spec, feat_spec, feat_spec,
                  full((F, D)), full((F, D)), full((10, D))],
        out_specs=(pl.BlockSpec((BLK, 10), lambda i: (i, 0)),
                   pl.BlockSpec((BLK, 10), lambda i: (i, 0))),
        out_shape=(out, out),
    )(uf, pf, nf, umap, imap, w1b)


def _tcb_body(ue, pe, ne, hfp, hfn, w1a, b1, w2, out):
    i = pl.program_id(0)
    uip = ue[...][:, :D] * pe[...][:, :D]
    uin = ue[...][:, :D] * ne[...][:, :D]

    c_last = (((1,), (1,)), ((), ()))
    hp = lax.dot_general(uip, w1a[...], c_last,
                         preferred_element_type=jnp.float32)
    hp = jnp.maximum(hp + hfp[...] + b1[...], 0.0)
    hn = lax.dot_general(uin, w1a[...], c_last,
                         preferred_element_type=jnp.float32)
    hn = jnp.maximum(hn + hfn[...] + b1[...], 0.0)

    d = lax.dot_general(hp - hn, w2[...], c_last,
                        preferred_element_type=jnp.float32)
    part = jnp.sum(jnp.minimum(d, 0.0) - jnp.log1p(jnp.exp(-jnp.abs(d))))

    @pl.when(i == 0)
    def _():
        out[0, 0] = 0.0

    out[0, 0] += part


def _tcb(gout, hfp, hfn, w1a, b1, w2):
    grid = B // BLK2
    h_spec = pl.BlockSpec((BLK2, 10), lambda i: (i, 0))

    def full(shape):
        return pl.BlockSpec(shape, lambda i: tuple(0 for _ in shape))

    emb = lambda s: pl.BlockSpec((BLK2, 128),
                                 lambda i, s=s: (i + s * (B // BLK2), 0))
    return pl.pallas_call(
        _tcb_body,
        grid=(grid,),
        in_specs=[emb(0), emb(1), emb(2), h_spec, h_spec,
                  full((10, D)), full((1, 10)), full((1, 10))],
        out_specs=pl.BlockSpec((1, 1), lambda i: (0, 0),
                               memory_space=pltpu.SMEM),
        out_shape=jax.ShapeDtypeStruct((1, 1), jnp.float32),
    )(gout, gout, gout, hfp, hfn, w1a, b1, w2)


def kernel(user_batch, user_feature_batch, pos_item_batch,
           pos_item_feature_batch, neg_item_batch, neg_item_feature_batch,
           user_emb, item_emb, user_map, item_map,
           fc1_w, fc1_b, fc2_w, fc2_b):
    uidx = user_batch.astype(jnp.int32)
    pidx = pos_item_batch.astype(jnp.int32)
    nidx = neg_item_batch.astype(jnp.int32)

    tail_u = jnp.zeros((D, 128), jnp.float32).at[:, :U - LANE_END].set(
        user_emb[LANE_END:].T)
    tail_i = jnp.zeros((D, 128), jnp.float32).at[:, :U - LANE_END].set(
        item_emb[LANE_END:].T)

    gout = _sc_gather(uidx, pidx, nidx, user_emb.T, item_emb.T,
                      tail_u, tail_i)[0]
    hfp, hfn = _tca(user_feature_batch, pos_item_feature_batch,
                    neg_item_feature_batch, user_map, item_map,
                    fc1_w[:, D:])
    total = _tcb(gout, hfp, hfn, fc1_w[:, :D],
                 fc1_b.reshape(1, 10), fc2_w)
    return -total[0, 0] / B
